# baseline (speedup 1.0000x reference)
import functools

import jax
import jax.numpy as jnp
from jax import lax
from jax.experimental import pallas as pl
from jax.experimental.pallas import tpu as pltpu
from jax.experimental.pallas import tpu_sc as plsc

N = 10000
E = 320000
NC = 2
NS = 16
NW = NC * NS
EP = 327680
EPW = EP // NW
K = 80
NCH = EPW // K
NP = 10240
RPS = NP // NS
NB = 8
PD = 4

_MESH = plsc.VectorSubcoreMesh(
    core_axis_name="c", subcore_axis_name="s", num_cores=NC, num_subcores=NS)


def _edge_pass(D):
    def body(g_hbm, src_hbm, dst_hbm, z_hbm, out_hbm,
             src_v, dst_v, rows_v, acc_sh, *sems):
        c = lax.axis_index("c")
        s = lax.axis_index("s")
        w = c * NS + s
        pltpu.sync_copy(z_hbm.at[pl.ds(s * RPS, RPS)],
                        acc_sh.at[pl.ds(s * RPS, RPS)])
        pltpu.sync_copy(src_hbm.at[w], src_v)
        pltpu.sync_copy(dst_hbm.at[w], dst_v)
        plsc.subcore_barrier()

        def fire_g(j, b):
            pltpu.async_copy(g_hbm.at[src_v.at[j]], rows_v.at[b], sems[b])

        def wait_g(j, b):
            pltpu.make_async_copy(
                g_hbm.at[src_v.at[j]], rows_v.at[b], sems[b]).wait()

        for b in range(PD):
            fire_g(b, b)

        def outer(i, carry):
            for t in range(PD):
                j = i * PD + t
                wait_g(j, t)
                pltpu.sync_copy(rows_v.at[t], acc_sh.at[dst_v.at[j]],
                                add=True)
                jn = j + PD

                @pl.when(jn < NCH)
                def _():
                    fire_g(jn, t)
            return carry

        lax.fori_loop(0, NCH // PD, outer, 0)
        plsc.subcore_barrier()
        pltpu.sync_copy(acc_sh.at[pl.ds(s * RPS, RPS)],
                        out_hbm.at[c, pl.ds(s * RPS, RPS)])

    return pl.kernel(
        body,
        out_type=jax.ShapeDtypeStruct((NC, NP, D), jnp.float32),
        mesh=_MESH,
        compiler_params=pltpu.CompilerParams(use_tc_tiling_on_sc=False),
        scratch_types=[
            pltpu.VMEM((NCH, K), jnp.int32),
            pltpu.VMEM((NCH, K), jnp.int32),
            pltpu.VMEM((PD, K, D), jnp.float32),
            pltpu.VMEM_SHARED((NP, D), jnp.float32),
        ] + [pltpu.SemaphoreType.DMA] * PD,
    )


def _deg_pass():
    def body(dst_hbm, z_hbm, ones_hbm, out_hbm, dst_v, ones_v, acc_sh):
        c = lax.axis_index("c")
        s = lax.axis_index("s")
        w = c * NS + s
        pltpu.sync_copy(z_hbm.at[pl.ds(s * RPS, RPS)],
                        acc_sh.at[pl.ds(s * RPS, RPS)])
        pltpu.sync_copy(dst_hbm.at[w], dst_v)
        pltpu.sync_copy(ones_hbm, ones_v)
        plsc.subcore_barrier()

        def chunk(j, carry):
            pltpu.sync_copy(ones_v, acc_sh.at[dst_v.at[j]], add=True)
            return carry

        lax.fori_loop(0, NCH, chunk, 0)
        plsc.subcore_barrier()
        pltpu.sync_copy(acc_sh.at[pl.ds(s * RPS, RPS)],
                        out_hbm.at[c, pl.ds(s * RPS, RPS)])

    return pl.kernel(
        body,
        out_type=jax.ShapeDtypeStruct((NC, NP, 16), jnp.float32),
        mesh=_MESH,
        compiler_params=pltpu.CompilerParams(use_tc_tiling_on_sc=False),
        scratch_types=[
            pltpu.VMEM((NCH, K), jnp.int32),
            pltpu.VMEM((K, 16), jnp.float32),
            pltpu.VMEM_SHARED((NP, 16), jnp.float32),
        ],
    )


def _tc1_body(degp, x, w1, dinv_o, g1_o):
    deg = degp[0, :N, 0:1] + degp[1, :N, 0:1] + 1.0
    dinv = lax.rsqrt(deg)
    dinv_o[...] = dinv
    g1_o[...] = dinv * jnp.dot(x[...], w1[...],
                               preferred_element_type=jnp.float32)


def _tc2_body(accp, g1, dinv, b1, w2, x1_o, g2_o):
    d = dinv[...]
    x1 = jnp.maximum(d * (accp[0, :N] + accp[1, :N] + g1[...]) + b1[...], 0.0)
    x1_o[...] = x1
    g2_o[...] = d * jnp.dot(x1, w2[...], preferred_element_type=jnp.float32)


def _tc3_body(accp, g2, dinv, b2, x, ws02, bs02, w3, g3_o):
    d = dinv[...]
    x2 = jnp.maximum(d * (accp[0, :N] + accp[1, :N] + g2[...]) + b2[...], 0.0)
    x2 = x2 + jnp.dot(x[...], ws02[...],
                      preferred_element_type=jnp.float32) + bs02[...]
    g3_o[...] = d * jnp.dot(x2, w3[...], preferred_element_type=jnp.float32)


def _tc4_body(accp, g3, dinv, b3, x, ws03, bs03, x1, ws13, bs13, out_o):
    d = dinv[...]
    x3 = jnp.maximum(d * (accp[0, :N] + accp[1, :N] + g3[...]) + b3[...], 0.0)
    x3 = x3 + jnp.dot(x[...], ws03[...],
                      preferred_element_type=jnp.float32) + bs03[...]
    x3 = x3 + jnp.dot(x1[...], ws13[...],
                      preferred_element_type=jnp.float32) + bs13[...]
    out_o[...] = x3


def _tc(body, out_shapes):
    return pl.pallas_call(
        body,
        out_shape=[jax.ShapeDtypeStruct(s, jnp.float32) for s in out_shapes])


@jax.jit
def kernel(x, edge_index, W1, b1, W2, b2, W3, b3,
           Ws02, bs02, Ws03, bs03, Ws13, bs13):
    ei = edge_index.astype(jnp.int32)
    pad_src = jnp.zeros((EP - E,), jnp.int32)
    pad_dst = N + jnp.arange(EP - E, dtype=jnp.int32) % (NP - N)
    src = jnp.concatenate([ei[0], pad_src]).reshape(NW, NCH, K)
    dst = jnp.concatenate([ei[1], pad_dst]).reshape(NW, NCH, K)

    z64 = jnp.zeros((NP, 64), jnp.float32)
    z32 = jnp.zeros((NP, 32), jnp.float32)
    z16 = jnp.zeros((NP, 16), jnp.float32)
    ones = jnp.ones((K, 16), jnp.float32)

    degp = _deg_pass()(dst, z16, ones)
    dinv, g1 = _tc(_tc1_body, [(N, 1), (N, 64)])(degp, x, W1)

    acc1 = _edge_pass(64)(g1, src, dst, z64)
    x1, g2 = _tc(_tc2_body, [(N, 64), (N, 32)])(
        acc1, g1, dinv, b1.reshape(1, -1), W2)

    acc2 = _edge_pass(32)(g2, src, dst, z32)
    g3, = _tc(_tc3_body, [(N, 16)])(
        acc2, g2, dinv, b2.reshape(1, -1), x, Ws02, bs02.reshape(1, -1), W3)

    acc3 = _edge_pass(16)(g3, src, dst, z16)
    out, = _tc(_tc4_body, [(N, 16)])(
        acc3, g3, dinv, b3.reshape(1, -1), x, Ws03, bs03.reshape(1, -1),
        x1, Ws13, bs13.reshape(1, -1))
    return out


# speedup vs baseline: 2.1373x; 2.1373x over previous
import functools

import jax
import jax.numpy as jnp
from jax import lax
from jax.experimental import pallas as pl
from jax.experimental.pallas import tpu as pltpu
from jax.experimental.pallas import tpu_sc as plsc

N = 10000
E = 320000
NC = 2
NS = 16
NW = NC * NS
EPW = E // NW
K = 80
NCH = EPW // K
NP = 10240
RPS = NP // NS
NB = 8
PD = 5

_MESH = plsc.VectorSubcoreMesh(
    core_axis_name="c", subcore_axis_name="s", num_cores=NC, num_subcores=NS)


def _edge_pass(D):
    def body(g_hbm, src_hbm, dst_hbm, z_hbm, out_hbm,
             src_v, dst_v, rows_v, acc_sh, *sems):
        c = lax.axis_index("c")
        s = lax.axis_index("s")
        w = c * NS + s
        pltpu.sync_copy(z_hbm.at[pl.ds(s * RPS, RPS)],
                        acc_sh.at[pl.ds(s * RPS, RPS)])
        pltpu.sync_copy(src_hbm.at[w], src_v)
        pltpu.sync_copy(dst_hbm.at[w], dst_v)
        plsc.subcore_barrier()

        def fire_g(j, b):
            pltpu.async_copy(g_hbm.at[src_v.at[j]], rows_v.at[b], sems[b])

        def wait_g(j, b):
            pltpu.make_async_copy(
                g_hbm.at[src_v.at[j]], rows_v.at[b], sems[b]).wait()

        for b in range(PD):
            fire_g(b, b)

        def outer(i, carry):
            for t in range(PD):
                j = i * PD + t
                wait_g(j, t)
                pltpu.sync_copy(rows_v.at[t], acc_sh.at[dst_v.at[j]],
                                add=True)
                jn = j + PD

                @pl.when(jn < NCH)
                def _():
                    fire_g(jn, t)
            return carry

        lax.fori_loop(0, NCH // PD, outer, 0)
        plsc.subcore_barrier()
        pltpu.sync_copy(acc_sh.at[pl.ds(s * RPS, RPS)],
                        out_hbm.at[c, pl.ds(s * RPS, RPS)])

    return pl.kernel(
        body,
        out_type=jax.ShapeDtypeStruct((NC, NP, D), jnp.float32),
        mesh=_MESH,
        compiler_params=pltpu.CompilerParams(use_tc_tiling_on_sc=False),
        scratch_types=[
            pltpu.VMEM((NCH, K), jnp.int32),
            pltpu.VMEM((NCH, K), jnp.int32),
            pltpu.VMEM((PD, K, D), jnp.float32),
            pltpu.VMEM_SHARED((NP, D), jnp.float32),
        ] + [pltpu.SemaphoreType.DMA] * PD,
    )


def _deg_pass():
    def body(dst_hbm, z_hbm, ones_hbm, out_hbm, dst_v, ones_v, acc_sh):
        c = lax.axis_index("c")
        s = lax.axis_index("s")
        w = c * NS + s
        pltpu.sync_copy(z_hbm.at[pl.ds(s * RPS, RPS)],
                        acc_sh.at[pl.ds(s * RPS, RPS)])
        pltpu.sync_copy(dst_hbm.at[w], dst_v)
        pltpu.sync_copy(ones_hbm, ones_v)
        plsc.subcore_barrier()

        def chunk(j, carry):
            pltpu.sync_copy(ones_v, acc_sh.at[dst_v.at[j]], add=True)
            return carry

        lax.fori_loop(0, NCH, chunk, 0)
        plsc.subcore_barrier()
        pltpu.sync_copy(acc_sh.at[pl.ds(s * RPS, RPS)],
                        out_hbm.at[c, pl.ds(s * RPS, RPS)])

    return pl.kernel(
        body,
        out_type=jax.ShapeDtypeStruct((NC, NP, 16), jnp.float32),
        mesh=_MESH,
        compiler_params=pltpu.CompilerParams(use_tc_tiling_on_sc=False),
        scratch_types=[
            pltpu.VMEM((NCH, K), jnp.int32),
            pltpu.VMEM((K, 16), jnp.float32),
            pltpu.VMEM_SHARED((NP, 16), jnp.float32),
        ],
    )


def _tc1_body(degp, x, w1, dinv_o, g1_o):
    deg = degp[0, :N, 0:1] + degp[1, :N, 0:1] + 1.0
    dinv = lax.rsqrt(deg)
    dinv_o[...] = dinv
    g1_o[...] = dinv * jnp.dot(x[...], w1[...],
                               preferred_element_type=jnp.float32)


def _tc2_body(accp, g1, dinv, b1, w2, x1_o, g2_o):
    d = dinv[...]
    x1 = jnp.maximum(d * (accp[0, :N] + accp[1, :N] + g1[...]) + b1[...], 0.0)
    x1_o[...] = x1
    g2_o[...] = d * jnp.dot(x1, w2[...], preferred_element_type=jnp.float32)


def _tc3_body(accp, g2, dinv, b2, x, ws02, bs02, w3, g3_o):
    d = dinv[...]
    x2 = jnp.maximum(d * (accp[0, :N] + accp[1, :N] + g2[...]) + b2[...], 0.0)
    x2 = x2 + jnp.dot(x[...], ws02[...],
                      preferred_element_type=jnp.float32) + bs02[...]
    g3_o[...] = d * jnp.dot(x2, w3[...], preferred_element_type=jnp.float32)


def _tc4_body(accp, g3, dinv, b3, x, ws03, bs03, x1, ws13, bs13, out_o):
    d = dinv[...]
    x3 = jnp.maximum(d * (accp[0, :N] + accp[1, :N] + g3[...]) + b3[...], 0.0)
    x3 = x3 + jnp.dot(x[...], ws03[...],
                      preferred_element_type=jnp.float32) + bs03[...]
    x3 = x3 + jnp.dot(x1[...], ws13[...],
                      preferred_element_type=jnp.float32) + bs13[...]
    out_o[...] = x3


def _tc(body, out_shapes):
    return pl.pallas_call(
        body,
        out_shape=[jax.ShapeDtypeStruct(s, jnp.float32) for s in out_shapes])


@jax.jit
def kernel(x, edge_index, W1, b1, W2, b2, W3, b3,
           Ws02, bs02, Ws03, bs03, Ws13, bs13):
    ei = edge_index.astype(jnp.int32)
    src = ei[0].reshape(NW, NCH, K)
    dst = ei[1].reshape(NW, NCH, K)

    z64 = jnp.zeros((NP, 64), jnp.float32)
    z32 = jnp.zeros((NP, 32), jnp.float32)
    z16 = jnp.zeros((NP, 16), jnp.float32)
    ones = jnp.ones((K, 16), jnp.float32)

    degp = _deg_pass()(dst, z16, ones)
    dinv, g1 = _tc(_tc1_body, [(N, 1), (N, 64)])(degp, x, W1)

    acc1 = _edge_pass(64)(g1, src, dst, z64)
    x1, g2 = _tc(_tc2_body, [(N, 64), (N, 32)])(
        acc1, g1, dinv, b1.reshape(1, -1), W2)

    acc2 = _edge_pass(32)(g2, src, dst, z32)
    g3, = _tc(_tc3_body, [(N, 16)])(
        acc2, g2, dinv, b2.reshape(1, -1), x, Ws02, bs02.reshape(1, -1), W3)

    acc3 = _edge_pass(16)(g3, src, dst, z16)
    out, = _tc(_tc4_body, [(N, 16)])(
        acc3, g3, dinv, b3.reshape(1, -1), x, Ws03, bs03.reshape(1, -1),
        x1, Ws13, bs13.reshape(1, -1))
    return out


# speedup vs baseline: 2.1526x; 1.0072x over previous
import functools

import jax
import jax.numpy as jnp
from jax import lax
from jax.experimental import pallas as pl
from jax.experimental.pallas import tpu as pltpu
from jax.experimental.pallas import tpu_sc as plsc

N = 10000
E = 320000
NC = 2
NS = 16
NW = NC * NS
EPW = E // NW
K = 80
NCH = EPW // K
NP = 10240
RPS = NP // NS
NB = 8
PD = 5

_MESH = plsc.VectorSubcoreMesh(
    core_axis_name="c", subcore_axis_name="s", num_cores=NC, num_subcores=NS)


def _edge_pass(D):
    def body(g_hbm, src_hbm, dst_hbm, z_hbm, out_hbm,
             src_v, dst_v, rows_v, acc_sh, *sems):
        c = lax.axis_index("c")
        s = lax.axis_index("s")
        w = c * NS + s
        pltpu.sync_copy(z_hbm.at[pl.ds(s * RPS, RPS)],
                        acc_sh.at[pl.ds(s * RPS, RPS)])
        pltpu.sync_copy(src_hbm.at[w], src_v)
        pltpu.sync_copy(dst_hbm.at[w], dst_v)
        plsc.subcore_barrier()

        def fire_g(j, b):
            pltpu.async_copy(g_hbm.at[src_v.at[j]], rows_v.at[b], sems[b])

        def wait_g(j, b):
            pltpu.make_async_copy(
                g_hbm.at[src_v.at[j]], rows_v.at[b], sems[b]).wait()

        for b in range(PD):
            fire_g(b, b)

        def outer(i, carry):
            for t in range(PD):
                j = i * PD + t
                wait_g(j, t)
                pltpu.sync_copy(rows_v.at[t], acc_sh.at[dst_v.at[j]],
                                add=True)
                jn = j + PD

                @pl.when(jn < NCH)
                def _():
                    fire_g(jn, t)
            return carry

        lax.fori_loop(0, NCH // PD, outer, 0)
        plsc.subcore_barrier()
        pltpu.sync_copy(acc_sh.at[pl.ds(s * RPS, RPS)],
                        out_hbm.at[c, pl.ds(s * RPS, RPS)])

    return pl.kernel(
        body,
        out_type=jax.ShapeDtypeStruct((NC, NP, D), jnp.float32),
        mesh=_MESH,
        compiler_params=pltpu.CompilerParams(use_tc_tiling_on_sc=False),
        scratch_types=[
            pltpu.VMEM((NCH, K), jnp.int32),
            pltpu.VMEM((NCH, K), jnp.int32),
            pltpu.VMEM((PD, K, D), jnp.float32),
            pltpu.VMEM_SHARED((NP, D), jnp.float32),
        ] + [pltpu.SemaphoreType.DMA] * PD,
    )


def _deg_pass():
    def body(dst_hbm, z_hbm, ones_hbm, out_hbm, dst_v, ones_v, acc_sh):
        c = lax.axis_index("c")
        s = lax.axis_index("s")
        w = c * NS + s
        pltpu.sync_copy(z_hbm.at[pl.ds(s * RPS, RPS)],
                        acc_sh.at[pl.ds(s * RPS, RPS)])
        pltpu.sync_copy(dst_hbm.at[w], dst_v)
        pltpu.sync_copy(ones_hbm, ones_v)
        plsc.subcore_barrier()

        def chunk(j, carry):
            pltpu.sync_copy(ones_v, acc_sh.at[dst_v.at[j]], add=True)
            return carry

        lax.fori_loop(0, NCH, chunk, 0)
        plsc.subcore_barrier()
        pltpu.sync_copy(acc_sh.at[pl.ds(s * RPS, RPS)],
                        out_hbm.at[c, pl.ds(s * RPS, RPS)])

    return pl.kernel(
        body,
        out_type=jax.ShapeDtypeStruct((NC, NP, 16), jnp.float32),
        mesh=_MESH,
        compiler_params=pltpu.CompilerParams(use_tc_tiling_on_sc=False),
        scratch_types=[
            pltpu.VMEM((NCH, K), jnp.int32),
            pltpu.VMEM((K, 16), jnp.float32),
            pltpu.VMEM_SHARED((NP, 16), jnp.float32),
        ],
    )


R = 2000
GRID = N // R

def _rows(shape):
    return pl.BlockSpec((R,) + shape, lambda i: (i, 0))

def _accs(d):
    return pl.BlockSpec((2, R, d), lambda i: (0, i, 0))

def _full(shape):
    return pl.BlockSpec(shape, lambda i: (0,) * len(shape))


def _tc1_body(degp, x, w1, dinv_o, g1_o):
    deg = degp[0, :, 0:1] + degp[1, :, 0:1] + 1.0
    dinv = lax.rsqrt(deg)
    dinv_o[...] = dinv
    g1_o[...] = dinv * jnp.dot(x[...], w1[...],
                               preferred_element_type=jnp.float32)


def _tc2_body(accp, g1, dinv, b1, w2, x1_o, g2_o):
    d = dinv[...]
    x1 = jnp.maximum(d * (accp[0] + accp[1] + g1[...]) + b1[...], 0.0)
    x1_o[...] = x1
    g2_o[...] = d * jnp.dot(x1, w2[...], preferred_element_type=jnp.float32)


def _tc3_body(accp, g2, dinv, b2, x, ws02, bs02, w3, g3_o):
    d = dinv[...]
    x2 = jnp.maximum(d * (accp[0] + accp[1] + g2[...]) + b2[...], 0.0)
    x2 = x2 + jnp.dot(x[...], ws02[...],
                      preferred_element_type=jnp.float32) + bs02[...]
    g3_o[...] = d * jnp.dot(x2, w3[...], preferred_element_type=jnp.float32)


def _tc4_body(accp, g3, dinv, b3, x, ws03, bs03, x1, ws13, bs13, out_o):
    d = dinv[...]
    x3 = jnp.maximum(d * (accp[0] + accp[1] + g3[...]) + b3[...], 0.0)
    x3 = x3 + jnp.dot(x[...], ws03[...],
                      preferred_element_type=jnp.float32) + bs03[...]
    x3 = x3 + jnp.dot(x1[...], ws13[...],
                      preferred_element_type=jnp.float32) + bs13[...]
    out_o[...] = x3


def _tc(body, in_specs, out_shapes, out_specs):
    return pl.pallas_call(
        body,
        grid=(GRID,),
        in_specs=in_specs,
        out_specs=out_specs,
        out_shape=[jax.ShapeDtypeStruct(s, jnp.float32) for s in out_shapes])


@jax.jit
def kernel(x, edge_index, W1, b1, W2, b2, W3, b3,
           Ws02, bs02, Ws03, bs03, Ws13, bs13):
    ei = edge_index.astype(jnp.int32)
    src = ei[0].reshape(NW, NCH, K)
    dst = ei[1].reshape(NW, NCH, K)

    z64 = jnp.zeros((NP, 64), jnp.float32)
    z32 = jnp.zeros((NP, 32), jnp.float32)
    z16 = jnp.zeros((NP, 16), jnp.float32)
    ones = jnp.ones((K, 16), jnp.float32)

    degp = _deg_pass()(dst, z16, ones)
    dinv, g1 = _tc(
        _tc1_body,
        [_accs(16), _rows((128,)), _full((128, 64))],
        [(N, 1), (N, 64)],
        [_rows((1,)), _rows((64,))])(degp, x, W1)

    acc1 = _edge_pass(64)(g1, src, dst, z64)
    x1, g2 = _tc(
        _tc2_body,
        [_accs(64), _rows((64,)), _rows((1,)), _full((1, 64)),
         _full((64, 32))],
        [(N, 64), (N, 32)],
        [_rows((64,)), _rows((32,))])(
        acc1, g1, dinv, b1.reshape(1, -1), W2)

    acc2 = _edge_pass(32)(g2, src, dst, z32)
    g3, = _tc(
        _tc3_body,
        [_accs(32), _rows((32,)), _rows((1,)), _full((1, 32)),
         _rows((128,)), _full((128, 32)), _full((1, 32)), _full((32, 16))],
        [(N, 16)],
        [_rows((16,))])(
        acc2, g2, dinv, b2.reshape(1, -1), x, Ws02, bs02.reshape(1, -1), W3)

    acc3 = _edge_pass(16)(g3, src, dst, z16)
    out, = _tc(
        _tc4_body,
        [_accs(16), _rows((16,)), _rows((1,)), _full((1, 16)),
         _rows((128,)), _full((128, 16)), _full((1, 16)),
         _rows((64,)), _full((64, 16)), _full((1, 16))],
        [(N, 16)],
        [_rows((16,))])(
        acc3, g3, dinv, b3.reshape(1, -1), x, Ws03, bs03.reshape(1, -1),
        x1, Ws13, bs13.reshape(1, -1))
    return out
